# baseline (speedup 1.0000x reference)
import functools

import jax
import jax.numpy as jnp
from jax import lax
from jax.experimental import pallas as pl
from jax.experimental.pallas import tpu as pltpu
from jax.experimental.pallas import tpu_sc as plsc

N_NODES = 10000
F_OUT = 128
NUM_CORES = 2
NUM_SUBCORES = 16
NUM_WORKERS = NUM_CORES * NUM_SUBCORES
CHUNK = 80
LANES = 16
ZERO_ROWS = 80


def _pq_body(h_ref, r_ref, a0_ref, a1_ref, p_ref, q_ref):
    p_ref[...] = jnp.dot(h_ref[...], a0_ref[...],
                         preferred_element_type=jnp.float32)
    q_ref[...] = jnp.dot(r_ref[...], a1_ref[...],
                         preferred_element_type=jnp.float32)


COMBINE_BLK = 2000


def _combine_body(hp_ref, rs_ref, o_ref):
    rs = rs_ref[0, :, :] + rs_ref[1, :, :]
    o_ref[0, :, :] = (hp_ref[0, :, :] + hp_ref[1, :, :]) / rs


def _make_sc_kernel(n_edges):
    edges_per_worker = n_edges // NUM_WORKERS
    n_chunks = edges_per_worker // CHUNK
    n_zero_iters = -(-(N_NODES // ZERO_ROWS) // NUM_SUBCORES)
    mesh = plsc.VectorSubcoreMesh(core_axis_name="c", subcore_axis_name="s")

    @functools.partial(
        pl.kernel,
        out_type=[
            jax.ShapeDtypeStruct((NUM_CORES, N_NODES, F_OUT), jnp.float32),
            jax.ShapeDtypeStruct((NUM_CORES, N_NODES), jnp.float32),
        ],
        mesh=mesh,
        compiler_params=pltpu.CompilerParams(needs_layout_passes=False),
        scratch_types=[
            pltpu.VMEM((CHUNK, F_OUT), jnp.float32),
            pltpu.VMEM((CHUNK, F_OUT), jnp.float32),
            pltpu.VMEM((CHUNK, F_OUT), jnp.float32),
            pltpu.VMEM((CHUNK, F_OUT), jnp.float32),
            pltpu.VMEM((3, CHUNK), jnp.int32),
            pltpu.VMEM((3, CHUNK), jnp.int32),
            pltpu.VMEM((CHUNK + LANES,), jnp.float32),
            pltpu.VMEM((CHUNK + LANES,), jnp.float32),
            pltpu.VMEM((CHUNK,), jnp.float32),
            pltpu.VMEM((CHUNK,), jnp.float32),
            pltpu.VMEM((CHUNK,), jnp.float32),
            pltpu.VMEM((CHUNK,), jnp.float32),
            pltpu.VMEM((CHUNK,), jnp.int32),
            pltpu.VMEM((CHUNK,), jnp.int32),
            pltpu.VMEM_SHARED((N_NODES, F_OUT), jnp.float32),
            pltpu.VMEM_SHARED((N_NODES,), jnp.float32),
            pltpu.SemaphoreType.DMA,
            pltpu.SemaphoreType.DMA,
            pltpu.SemaphoreType.DMA,
            pltpu.SemaphoreType.DMA,
            pltpu.SemaphoreType.DMA,
            pltpu.SemaphoreType.DMA,
            pltpu.SemaphoreType.DMA,
            pltpu.SemaphoreType.DMA,
            pltpu.SemaphoreType.DMA,
            pltpu.SemaphoreType.DMA,
            pltpu.SemaphoreType.DMA,
            pltpu.SemaphoreType.DMA,
            pltpu.SemaphoreType.DMA,
            pltpu.SemaphoreType.DMA,
        ],
    )
    def sc_kernel(h_hbm, r_hbm, p_hbm, q_hbm, ablk_hbm,
                  hp_out, rs_out,
                  bufH0, bufH1, bufR0, bufR1, idx0, idx1, eb0, eb1,
                  pb0, pb1, qb0, qb1, dstS0, dstS1, hp_shared, rs_shared,
                  s_idx0, s_idx1, s_h0, s_h1, s_r0, s_r1,
                  s_rs0, s_rs1, s_hp0, s_hp1, s_p0, s_p1, s_q0, s_q1):
        cid = lax.axis_index("c")
        sid = lax.axis_index("s")
        wid = cid * NUM_SUBCORES + sid

        bufH = [bufH0, bufH1]
        bufR = [bufR0, bufR1]
        idx3 = [idx0, idx1]
        ebuf = [eb0, eb1]
        pb = [pb0, pb1]
        qb = [qb0, qb1]
        dstS = [dstS0, dstS1]
        s_idx = [s_idx0, s_idx1]
        s_h = [s_h0, s_h1]
        s_r = [s_r0, s_r1]
        s_rs = [s_rs0, s_rs1]
        s_hp = [s_hp0, s_hp1]
        s_p = [s_p0, s_p1]
        s_q = [s_q0, s_q1]

        zero16f = jnp.zeros((LANES,), jnp.float32)

        def _zero_row(i, _):
            for c in range(F_OUT // LANES):
                bufH0[i, pl.ds(c * LANES, LANES)] = zero16f
            return 0
        lax.fori_loop(0, CHUNK, _zero_row, 0)
        for i in range((CHUNK + LANES) // LANES):
            eb0[pl.ds(i * LANES, LANES)] = zero16f

        for t in range(n_zero_iters):
            k = sid + NUM_SUBCORES * t

            @pl.when(k * ZERO_ROWS < N_NODES)
            def _():
                pltpu.sync_copy(bufH0,
                                hp_shared.at[pl.ds(k * ZERO_ROWS, ZERO_ROWS), :])
                pltpu.sync_copy(eb0.at[pl.ds(0, ZERO_ROWS)],
                                rs_shared.at[pl.ds(k * ZERO_ROWS, ZERO_ROWS)])

        plsc.subcore_barrier()

        def _idx_cp(p, c):
            return pltpu.make_async_copy(
                ablk_hbm.at[wid * n_chunks + c], idx3[p], s_idx[p])

        def _hp_cp(p):
            return pltpu.make_async_copy(
                bufH[p], hp_shared.at[dstS[p]], s_hp[p])

        def _rs_cp(p):
            return pltpu.make_async_copy(
                ebuf[p].at[pl.ds(LANES, CHUNK)],
                rs_shared.at[dstS[p]], s_rs[p])

        def _m_compute(p):
            def grp(gi, _):
                j0 = gi * LANES
                for k in range(LANES):
                    j = j0 + k
                    ej = plsc.load_gather(
                        ebuf[p], [jnp.full((LANES,), LANES, jnp.int32) + j])
                    for c in range(F_OUT // LANES):
                        sl = pl.ds(c * LANES, LANES)
                        bufH[p][j, sl] = (bufH[p][j, sl] - bufR[p][j, sl]) * ej
                return 0
            lax.fori_loop(0, CHUNK // LANES, grp, 0)

        def step(p, c):
            q = 1 - p
            @pl.when(c >= 2)
            def _():
                _hp_cp(p).wait()
                _rs_cp(p).wait()

            @pl.when(c < n_chunks)
            def _():
                _idx_cp(p, c).wait()
                pltpu.async_copy(h_hbm.at[idx3[p].at[2]], bufH[p], s_h[p])
                pltpu.async_copy(r_hbm.at[idx3[p].at[1]], bufR[p], s_r[p])
                pltpu.async_copy(p_hbm.at[idx3[p].at[2]], pb[p], s_p[p])
                pltpu.async_copy(q_hbm.at[idx3[p].at[1]], qb[p], s_q[p])
                for i in range(CHUNK // LANES):
                    sl = pl.ds(i * LANES, LANES)
                    dstS[p][sl] = idx3[p][0, sl]

            @pl.when(c + 1 < n_chunks)
            def _():
                _idx_cp(q, c + 1).start()

            @pl.when(c >= 1)
            def _():
                _m_compute(q)
                pltpu.async_copy(bufH[q], hp_shared.at[dstS[q]],
                                 s_hp[q], add=True)

            @pl.when(c < n_chunks)
            def _():
                pltpu.make_async_copy(
                    p_hbm.at[idx3[p].at[2]], pb[p], s_p[p]).wait()
                pltpu.make_async_copy(
                    q_hbm.at[idx3[p].at[1]], qb[p], s_q[p]).wait()
                for i in range(CHUNK // LANES):
                    sl = pl.ds(i * LANES, LANES)
                    x = pb[p][sl] + qb[p][sl]
                    xl = jnp.where(x >= 0, x, 0.2 * x)
                    ebuf[p][pl.ds(LANES + i * LANES, LANES)] = jnp.exp(-xl)
                pltpu.async_copy(ebuf[p].at[pl.ds(LANES, CHUNK)],
                                 rs_shared.at[dstS[p]], s_rs[p], add=True)
                pltpu.make_async_copy(
                    h_hbm.at[idx3[p].at[2]], bufH[p], s_h[p]).wait()
                pltpu.make_async_copy(
                    r_hbm.at[idx3[p].at[1]], bufR[p], s_r[p]).wait()

        def pair(t, _):
            step(0, 2 * t)
            step(1, 2 * t + 1)
            return 0

        _idx_cp(0, 0).start()
        lax.fori_loop(0, (n_chunks + 2) // 2, pair, 0)
        last = (n_chunks - 1) % 2
        _hp_cp(last).wait()
        _rs_cp(last).wait()

        plsc.subcore_barrier()

        row0 = pl.multiple_of(sid * 624, 8)
        pltpu.sync_copy(hp_shared.at[pl.ds(row0, 624), :],
                        hp_out.at[cid, pl.ds(row0, 624), :])

        @pl.when(sid == NUM_SUBCORES - 1)
        def _():
            tail = NUM_SUBCORES * 624
            pltpu.sync_copy(hp_shared.at[pl.ds(tail, N_NODES - tail), :],
                            hp_out.at[cid, pl.ds(tail, N_NODES - tail), :])

        @pl.when(sid == 0)
        def _():
            pltpu.sync_copy(rs_shared, rs_out.at[cid])

    return sc_kernel


def kernel(h, inputr, A, a_src_dst):
    n_nodes, f_out = h.shape
    n_edges = A.shape[1]
    a0 = a_src_dst[0, 0]
    a1 = a_src_dst[0, 1]

    p, q = pl.pallas_call(
        _pq_body,
        out_shape=[
            jax.ShapeDtypeStruct((n_nodes, 1), jnp.float32),
            jax.ShapeDtypeStruct((inputr.shape[0], 1), jnp.float32),
        ],
    )(h, inputr, a0, a1)
    p = p.reshape(n_nodes)
    q = q.reshape(inputr.shape[0])

    n_chunks_total = n_edges // CHUNK
    a_blk = jnp.transpose(A.reshape(3, n_chunks_total, CHUNK), (1, 0, 2))

    hp_part, rs_part = _make_sc_kernel(n_edges)(h, inputr, p, q, a_blk)

    blk = COMBINE_BLK
    out = pl.pallas_call(
        _combine_body,
        grid=(n_nodes // blk,),
        in_specs=[
            pl.BlockSpec((2, blk, f_out), lambda i: (0, i, 0)),
            pl.BlockSpec((2, blk, 1), lambda i: (0, i, 0)),
        ],
        out_specs=pl.BlockSpec((1, blk, f_out), lambda i: (0, i, 0)),
        out_shape=jax.ShapeDtypeStruct((1, n_nodes, f_out), jnp.float32),
    )(hp_part, rs_part.reshape(2, n_nodes, 1))
    return out


# speedup vs baseline: 1.0857x; 1.0857x over previous
import functools

import jax
import jax.numpy as jnp
from jax import lax
from jax.experimental import pallas as pl
from jax.experimental.pallas import tpu as pltpu
from jax.experimental.pallas import tpu_sc as plsc

N_NODES = 10000
F_OUT = 128
NUM_CORES = 2
NUM_SUBCORES = 16
NUM_WORKERS = NUM_CORES * NUM_SUBCORES
CHUNK = 80
LANES = 16
ZERO_ROWS = 80


def _pq_body(h_ref, r_ref, a0_ref, a1_ref, p_ref, q_ref):
    p_ref[...] = jnp.dot(h_ref[...], a0_ref[...],
                         preferred_element_type=jnp.float32)
    q_ref[...] = jnp.dot(r_ref[...], a1_ref[...],
                         preferred_element_type=jnp.float32)


def _combine_body(hp_ref, rs_ref, o_ref):
    rs = rs_ref[0, :] + rs_ref[1, :]
    o_ref[0, :, :] = (hp_ref[0, :, :] + hp_ref[1, :, :]) / rs[:, None]


def _make_sc_kernel(n_edges):
    edges_per_worker = n_edges // NUM_WORKERS
    n_chunks = edges_per_worker // CHUNK
    n_zero_iters = -(-(N_NODES // ZERO_ROWS) // NUM_SUBCORES)
    mesh = plsc.VectorSubcoreMesh(core_axis_name="c", subcore_axis_name="s")

    @functools.partial(
        pl.kernel,
        out_type=[
            jax.ShapeDtypeStruct((NUM_CORES, N_NODES, F_OUT), jnp.float32),
            jax.ShapeDtypeStruct((NUM_CORES, N_NODES), jnp.float32),
        ],
        mesh=mesh,
        compiler_params=pltpu.CompilerParams(needs_layout_passes=False),
        scratch_types=[
            pltpu.VMEM((CHUNK, F_OUT), jnp.float32),
            pltpu.VMEM((CHUNK, F_OUT), jnp.float32),
            pltpu.VMEM((CHUNK, F_OUT), jnp.float32),
            pltpu.VMEM((CHUNK, F_OUT), jnp.float32),
            pltpu.VMEM((3, CHUNK), jnp.int32),
            pltpu.VMEM((3, CHUNK), jnp.int32),
            pltpu.VMEM((CHUNK + LANES,), jnp.float32),
            pltpu.VMEM((CHUNK + LANES,), jnp.float32),
            pltpu.VMEM((CHUNK,), jnp.float32),
            pltpu.VMEM((CHUNK,), jnp.float32),
            pltpu.VMEM((CHUNK,), jnp.float32),
            pltpu.VMEM((CHUNK,), jnp.float32),
            pltpu.VMEM((CHUNK,), jnp.int32),
            pltpu.VMEM((CHUNK,), jnp.int32),
            pltpu.VMEM_SHARED((N_NODES, F_OUT), jnp.float32),
            pltpu.VMEM_SHARED((N_NODES,), jnp.float32),
            pltpu.SemaphoreType.DMA,
            pltpu.SemaphoreType.DMA,
            pltpu.SemaphoreType.DMA,
            pltpu.SemaphoreType.DMA,
            pltpu.SemaphoreType.DMA,
            pltpu.SemaphoreType.DMA,
            pltpu.SemaphoreType.DMA,
            pltpu.SemaphoreType.DMA,
            pltpu.SemaphoreType.DMA,
            pltpu.SemaphoreType.DMA,
            pltpu.SemaphoreType.DMA,
            pltpu.SemaphoreType.DMA,
            pltpu.SemaphoreType.DMA,
            pltpu.SemaphoreType.DMA,
        ],
    )
    def sc_kernel(h_hbm, r_hbm, p_hbm, q_hbm, ablk_hbm,
                  hp_out, rs_out,
                  bufH0, bufH1, bufR0, bufR1, idx0, idx1, eb0, eb1,
                  pb0, pb1, qb0, qb1, dstS0, dstS1, hp_shared, rs_shared,
                  s_idx0, s_idx1, s_h0, s_h1, s_r0, s_r1,
                  s_rs0, s_rs1, s_hp0, s_hp1, s_p0, s_p1, s_q0, s_q1):
        cid = lax.axis_index("c")
        sid = lax.axis_index("s")
        wid = cid * NUM_SUBCORES + sid

        bufH = [bufH0, bufH1]
        bufR = [bufR0, bufR1]
        idx3 = [idx0, idx1]
        ebuf = [eb0, eb1]
        pb = [pb0, pb1]
        qb = [qb0, qb1]
        dstS = [dstS0, dstS1]
        s_idx = [s_idx0, s_idx1]
        s_h = [s_h0, s_h1]
        s_r = [s_r0, s_r1]
        s_rs = [s_rs0, s_rs1]
        s_hp = [s_hp0, s_hp1]
        s_p = [s_p0, s_p1]
        s_q = [s_q0, s_q1]

        zero16f = jnp.zeros((LANES,), jnp.float32)

        def _zero_row(i, _):
            for c in range(F_OUT // LANES):
                bufH0[i, pl.ds(c * LANES, LANES)] = zero16f
            return 0
        lax.fori_loop(0, CHUNK, _zero_row, 0)
        for i in range((CHUNK + LANES) // LANES):
            eb0[pl.ds(i * LANES, LANES)] = zero16f

        for t in range(n_zero_iters):
            k = sid + NUM_SUBCORES * t

            @pl.when(k * ZERO_ROWS < N_NODES)
            def _():
                pltpu.sync_copy(bufH0,
                                hp_shared.at[pl.ds(k * ZERO_ROWS, ZERO_ROWS), :])
                pltpu.sync_copy(eb0.at[pl.ds(0, ZERO_ROWS)],
                                rs_shared.at[pl.ds(k * ZERO_ROWS, ZERO_ROWS)])

        plsc.subcore_barrier()

        n_chunks_total = n_edges // CHUNK

        def _idx_start(p, c):
            base = wid * n_chunks + c
            for j in range(3):
                pltpu.async_copy(ablk_hbm.at[j * n_chunks_total + base],
                                 idx3[p].at[j], s_idx[p])

        def _idx_wait(p):
            for j in range(3):
                pltpu.make_async_copy(ablk_hbm.at[0], idx3[p].at[j],
                                      s_idx[p]).wait()

        def _hp_cp(p):
            return pltpu.make_async_copy(
                bufH[p], hp_shared.at[dstS[p]], s_hp[p])

        def _rs_cp(p):
            return pltpu.make_async_copy(
                ebuf[p].at[pl.ds(LANES, CHUNK)],
                rs_shared.at[dstS[p]], s_rs[p])

        def _m_compute(p):
            def grp(gi, _):
                j0 = gi * LANES
                for k in range(LANES):
                    j = j0 + k
                    ej = plsc.load_gather(
                        ebuf[p], [jnp.full((LANES,), LANES, jnp.int32) + j])
                    for c in range(F_OUT // LANES):
                        sl = pl.ds(c * LANES, LANES)
                        bufH[p][j, sl] = (bufH[p][j, sl] - bufR[p][j, sl]) * ej
                return 0
            lax.fori_loop(0, CHUNK // LANES, grp, 0)

        def step(p, c):
            q = 1 - p
            @pl.when(c >= 2)
            def _():
                _hp_cp(p).wait()
                _rs_cp(p).wait()

            @pl.when(c < n_chunks)
            def _():
                _idx_wait(p)
                pltpu.async_copy(h_hbm.at[idx3[p].at[2]], bufH[p], s_h[p])
                pltpu.async_copy(r_hbm.at[idx3[p].at[1]], bufR[p], s_r[p])
                pltpu.async_copy(p_hbm.at[idx3[p].at[2]], pb[p], s_p[p])
                pltpu.async_copy(q_hbm.at[idx3[p].at[1]], qb[p], s_q[p])
                for i in range(CHUNK // LANES):
                    sl = pl.ds(i * LANES, LANES)
                    dstS[p][sl] = idx3[p][0, sl]

            @pl.when(c + 1 < n_chunks)
            def _():
                _idx_start(q, c + 1)

            @pl.when(c >= 1)
            def _():
                _m_compute(q)
                pltpu.async_copy(bufH[q], hp_shared.at[dstS[q]],
                                 s_hp[q], add=True)

            @pl.when(c < n_chunks)
            def _():
                pltpu.make_async_copy(
                    p_hbm.at[idx3[p].at[2]], pb[p], s_p[p]).wait()
                pltpu.make_async_copy(
                    q_hbm.at[idx3[p].at[1]], qb[p], s_q[p]).wait()
                for i in range(CHUNK // LANES):
                    sl = pl.ds(i * LANES, LANES)
                    x = pb[p][sl] + qb[p][sl]
                    xl = jnp.where(x >= 0, x, 0.2 * x)
                    ebuf[p][pl.ds(LANES + i * LANES, LANES)] = jnp.exp(-xl)
                pltpu.async_copy(ebuf[p].at[pl.ds(LANES, CHUNK)],
                                 rs_shared.at[dstS[p]], s_rs[p], add=True)
                pltpu.make_async_copy(
                    h_hbm.at[idx3[p].at[2]], bufH[p], s_h[p]).wait()
                pltpu.make_async_copy(
                    r_hbm.at[idx3[p].at[1]], bufR[p], s_r[p]).wait()

        def pair(t, _):
            step(0, 2 * t)
            step(1, 2 * t + 1)
            return 0

        _idx_start(0, 0)
        lax.fori_loop(0, (n_chunks + 2) // 2, pair, 0)
        last = (n_chunks - 1) % 2
        _hp_cp(last).wait()
        _rs_cp(last).wait()

        plsc.subcore_barrier()

        row0 = pl.multiple_of(sid * 624, 8)
        pltpu.sync_copy(hp_shared.at[pl.ds(row0, 624), :],
                        hp_out.at[cid, pl.ds(row0, 624), :])

        @pl.when(sid == NUM_SUBCORES - 1)
        def _():
            tail = NUM_SUBCORES * 624
            pltpu.sync_copy(hp_shared.at[pl.ds(tail, N_NODES - tail), :],
                            hp_out.at[cid, pl.ds(tail, N_NODES - tail), :])

        @pl.when(sid == 0)
        def _():
            pltpu.sync_copy(rs_shared, rs_out.at[cid])

    return sc_kernel


def kernel(h, inputr, A, a_src_dst):
    n_nodes, f_out = h.shape
    n_edges = A.shape[1]
    a0 = a_src_dst[0, 0]
    a1 = a_src_dst[0, 1]

    p, q = pl.pallas_call(
        _pq_body,
        out_shape=[
            jax.ShapeDtypeStruct((n_nodes, 1), jnp.float32),
            jax.ShapeDtypeStruct((inputr.shape[0], 1), jnp.float32),
        ],
    )(h, inputr, a0, a1)
    p = p.reshape(n_nodes)
    q = q.reshape(inputr.shape[0])

    n_chunks_total = n_edges // CHUNK
    a_rows = A.reshape(3 * n_chunks_total, CHUNK)

    hp_part, rs_part = _make_sc_kernel(n_edges)(h, inputr, p, q, a_rows)

    out = pl.pallas_call(
        _combine_body,
        out_shape=jax.ShapeDtypeStruct((1, n_nodes, f_out), jnp.float32),
    )(hp_part, rs_part)
    return out


# speedup vs baseline: 1.0895x; 1.0035x over previous
import functools

import jax
import jax.numpy as jnp
from jax import lax
from jax.experimental import pallas as pl
from jax.experimental.pallas import tpu as pltpu
from jax.experimental.pallas import tpu_sc as plsc

N_NODES = 10000
F_OUT = 128
NUM_CORES = 2
NUM_SUBCORES = 16
NUM_WORKERS = NUM_CORES * NUM_SUBCORES
CHUNK = 80
LANES = 16
ZERO_ROWS = 80


def _pq_body(h_ref, r_ref, a0_ref, a1_ref, p_ref, q_ref):
    p_ref[...] = jnp.dot(h_ref[...], a0_ref[...],
                         preferred_element_type=jnp.float32)
    q_ref[...] = jnp.dot(r_ref[...], a1_ref[...],
                         preferred_element_type=jnp.float32)


def _combine_body(hp_ref, rs_ref, o_ref):
    rs = rs_ref[0, :] + rs_ref[1, :]
    o_ref[0, :, :] = (hp_ref[0, :, :] + hp_ref[1, :, :]) / rs[:, None]


def _make_sc_kernel(n_edges):
    edges_per_worker = n_edges // NUM_WORKERS
    n_chunks = edges_per_worker // CHUNK
    n_zero_iters = -(-(N_NODES // ZERO_ROWS) // NUM_SUBCORES)
    mesh = plsc.VectorSubcoreMesh(core_axis_name="c", subcore_axis_name="s")

    @functools.partial(
        pl.kernel,
        out_type=[
            jax.ShapeDtypeStruct((NUM_CORES, N_NODES, F_OUT), jnp.float32),
            jax.ShapeDtypeStruct((NUM_CORES, N_NODES), jnp.float32),
        ],
        mesh=mesh,
        compiler_params=pltpu.CompilerParams(needs_layout_passes=False),
        scratch_types=[
            pltpu.VMEM((CHUNK, F_OUT), jnp.float32),
            pltpu.VMEM((CHUNK, F_OUT), jnp.float32),
            pltpu.VMEM((CHUNK, F_OUT), jnp.float32),
            pltpu.VMEM((CHUNK, F_OUT), jnp.float32),
            pltpu.VMEM((3, CHUNK), jnp.int32),
            pltpu.VMEM((3, CHUNK), jnp.int32),
            pltpu.VMEM((CHUNK + LANES,), jnp.float32),
            pltpu.VMEM((CHUNK + LANES,), jnp.float32),
            pltpu.VMEM((CHUNK,), jnp.float32),
            pltpu.VMEM((CHUNK,), jnp.float32),
            pltpu.VMEM((CHUNK,), jnp.float32),
            pltpu.VMEM((CHUNK,), jnp.float32),
            pltpu.VMEM((CHUNK,), jnp.int32),
            pltpu.VMEM((CHUNK,), jnp.int32),
            pltpu.VMEM_SHARED((N_NODES, F_OUT), jnp.float32),
            pltpu.VMEM_SHARED((N_NODES,), jnp.float32),
            pltpu.SemaphoreType.DMA,
            pltpu.SemaphoreType.DMA,
            pltpu.SemaphoreType.DMA,
            pltpu.SemaphoreType.DMA,
            pltpu.SemaphoreType.DMA,
            pltpu.SemaphoreType.DMA,
            pltpu.SemaphoreType.DMA,
            pltpu.SemaphoreType.DMA,
            pltpu.SemaphoreType.DMA,
            pltpu.SemaphoreType.DMA,
            pltpu.SemaphoreType.DMA,
            pltpu.SemaphoreType.DMA,
            pltpu.SemaphoreType.DMA,
            pltpu.SemaphoreType.DMA,
        ],
    )
    def sc_kernel(h_hbm, r_hbm, p_hbm, q_hbm, ablk_hbm,
                  hp_out, rs_out,
                  bufH0, bufH1, bufR0, bufR1, idx0, idx1, eb0, eb1,
                  pb0, pb1, qb0, qb1, dstS0, dstS1, hp_shared, rs_shared,
                  s_idx0, s_idx1, s_h0, s_h1, s_r0, s_r1,
                  s_rs0, s_rs1, s_hp0, s_hp1, s_p0, s_p1, s_q0, s_q1):
        cid = lax.axis_index("c")
        sid = lax.axis_index("s")
        wid = cid * NUM_SUBCORES + sid

        bufH = [bufH0, bufH1]
        bufR = [bufR0, bufR1]
        idx3 = [idx0, idx1]
        ebuf = [eb0, eb1]
        pb = [pb0, pb1]
        qb = [qb0, qb1]
        dstS = [dstS0, dstS1]
        s_idx = [s_idx0, s_idx1]
        s_h = [s_h0, s_h1]
        s_r = [s_r0, s_r1]
        s_rs = [s_rs0, s_rs1]
        s_hp = [s_hp0, s_hp1]
        s_p = [s_p0, s_p1]
        s_q = [s_q0, s_q1]

        zero16f = jnp.zeros((LANES,), jnp.float32)

        def _zero_row(i, _):
            for c in range(F_OUT // LANES):
                bufH0[i, pl.ds(c * LANES, LANES)] = zero16f
            return 0
        lax.fori_loop(0, CHUNK, _zero_row, 0)
        for i in range((CHUNK + LANES) // LANES):
            eb0[pl.ds(i * LANES, LANES)] = zero16f

        for t in range(n_zero_iters):
            k = sid + NUM_SUBCORES * t

            @pl.when(k * ZERO_ROWS < N_NODES)
            def _():
                pltpu.async_copy(
                    bufH0, hp_shared.at[pl.ds(k * ZERO_ROWS, ZERO_ROWS), :],
                    s_h0)
                pltpu.async_copy(
                    eb0.at[pl.ds(0, ZERO_ROWS)],
                    rs_shared.at[pl.ds(k * ZERO_ROWS, ZERO_ROWS)], s_rs0)

        for t in range(n_zero_iters):
            k = sid + NUM_SUBCORES * t

            @pl.when(k * ZERO_ROWS < N_NODES)
            def _():
                pltpu.make_async_copy(
                    bufH0, hp_shared.at[pl.ds(k * ZERO_ROWS, ZERO_ROWS), :],
                    s_h0).wait()
                pltpu.make_async_copy(
                    eb0.at[pl.ds(0, ZERO_ROWS)],
                    rs_shared.at[pl.ds(k * ZERO_ROWS, ZERO_ROWS)],
                    s_rs0).wait()

        plsc.subcore_barrier()

        n_chunks_total = n_edges // CHUNK

        def _idx_start(p, c):
            base = wid * n_chunks + c
            for j in range(3):
                pltpu.async_copy(ablk_hbm.at[j * n_chunks_total + base],
                                 idx3[p].at[j], s_idx[p])

        def _idx_wait(p):
            for j in range(3):
                pltpu.make_async_copy(ablk_hbm.at[0], idx3[p].at[j],
                                      s_idx[p]).wait()

        def _hp_cp(p):
            return pltpu.make_async_copy(
                bufH[p], hp_shared.at[dstS[p]], s_hp[p])

        def _rs_cp(p):
            return pltpu.make_async_copy(
                ebuf[p].at[pl.ds(LANES, CHUNK)],
                rs_shared.at[dstS[p]], s_rs[p])

        def _m_compute(p):
            def grp(gi, _):
                j0 = gi * LANES
                for k in range(LANES):
                    j = j0 + k
                    ej = plsc.load_gather(
                        ebuf[p], [jnp.full((LANES,), LANES, jnp.int32) + j])
                    for c in range(F_OUT // LANES):
                        sl = pl.ds(c * LANES, LANES)
                        bufH[p][j, sl] = (bufH[p][j, sl] - bufR[p][j, sl]) * ej
                return 0
            lax.fori_loop(0, CHUNK // LANES, grp, 0)

        def step(p, c):
            q = 1 - p
            @pl.when(c >= 2)
            def _():
                _hp_cp(p).wait()
                _rs_cp(p).wait()

            @pl.when(c < n_chunks)
            def _():
                _idx_wait(p)
                pltpu.async_copy(h_hbm.at[idx3[p].at[2]], bufH[p], s_h[p])
                pltpu.async_copy(r_hbm.at[idx3[p].at[1]], bufR[p], s_r[p])
                pltpu.async_copy(p_hbm.at[idx3[p].at[2]], pb[p], s_p[p])
                pltpu.async_copy(q_hbm.at[idx3[p].at[1]], qb[p], s_q[p])
                for i in range(CHUNK // LANES):
                    sl = pl.ds(i * LANES, LANES)
                    dstS[p][sl] = idx3[p][0, sl]

            @pl.when(c + 1 < n_chunks)
            def _():
                _idx_start(q, c + 1)

            @pl.when(c >= 1)
            def _():
                _m_compute(q)
                pltpu.async_copy(bufH[q], hp_shared.at[dstS[q]],
                                 s_hp[q], add=True)

            @pl.when(c < n_chunks)
            def _():
                pltpu.make_async_copy(
                    p_hbm.at[idx3[p].at[2]], pb[p], s_p[p]).wait()
                pltpu.make_async_copy(
                    q_hbm.at[idx3[p].at[1]], qb[p], s_q[p]).wait()
                for i in range(CHUNK // LANES):
                    sl = pl.ds(i * LANES, LANES)
                    x = pb[p][sl] + qb[p][sl]
                    xl = jnp.where(x >= 0, x, 0.2 * x)
                    ebuf[p][pl.ds(LANES + i * LANES, LANES)] = jnp.exp(-xl)
                pltpu.async_copy(ebuf[p].at[pl.ds(LANES, CHUNK)],
                                 rs_shared.at[dstS[p]], s_rs[p], add=True)
                pltpu.make_async_copy(
                    h_hbm.at[idx3[p].at[2]], bufH[p], s_h[p]).wait()
                pltpu.make_async_copy(
                    r_hbm.at[idx3[p].at[1]], bufR[p], s_r[p]).wait()

        def pair(t, _):
            step(0, 2 * t)
            step(1, 2 * t + 1)
            return 0

        _idx_start(0, 0)
        lax.fori_loop(0, (n_chunks + 2) // 2, pair, 0)
        last = (n_chunks - 1) % 2
        _hp_cp(last).wait()
        _rs_cp(last).wait()

        plsc.subcore_barrier()

        row0 = pl.multiple_of(sid * 624, 8)
        pltpu.sync_copy(hp_shared.at[pl.ds(row0, 624), :],
                        hp_out.at[cid, pl.ds(row0, 624), :])

        @pl.when(sid == NUM_SUBCORES - 1)
        def _():
            tail = NUM_SUBCORES * 624
            pltpu.sync_copy(hp_shared.at[pl.ds(tail, N_NODES - tail), :],
                            hp_out.at[cid, pl.ds(tail, N_NODES - tail), :])

        @pl.when(sid == 0)
        def _():
            pltpu.sync_copy(rs_shared, rs_out.at[cid])

    return sc_kernel


def kernel(h, inputr, A, a_src_dst):
    n_nodes, f_out = h.shape
    n_edges = A.shape[1]
    a0 = a_src_dst[0, 0]
    a1 = a_src_dst[0, 1]

    p, q = pl.pallas_call(
        _pq_body,
        out_shape=[
            jax.ShapeDtypeStruct((n_nodes, 1), jnp.float32),
            jax.ShapeDtypeStruct((inputr.shape[0], 1), jnp.float32),
        ],
    )(h, inputr, a0, a1)
    p = p.reshape(n_nodes)
    q = q.reshape(inputr.shape[0])

    n_chunks_total = n_edges // CHUNK
    a_rows = A.reshape(3 * n_chunks_total, CHUNK)

    hp_part, rs_part = _make_sc_kernel(n_edges)(h, inputr, p, q, a_rows)

    out = pl.pallas_call(
        _combine_body,
        out_shape=jax.ShapeDtypeStruct((1, n_nodes, f_out), jnp.float32),
    )(hp_part, rs_part)
    return out
